# baseline (speedup 1.0000x reference)
import functools

import jax
import jax.numpy as jnp
from jax import lax
from jax.experimental import pallas as pl
from jax.experimental.pallas import tpu as pltpu
from jax.experimental.pallas import tpu_sc as plsc

BATCH = 4096
SEQ = 200
SEQ_PAD = 256
EMBED_DIM = 64
PAD_DIM = 128
VOCAB = 100000
ROWS = BATCH * SEQ
HALF = ROWS // 2
BHALF = BATCH // 2
NUM_CORES = 2
NUM_SUBCORES = 16
NW = NUM_CORES * NUM_SUBCORES
PPW = BHALF // NW
NTURN = PPW // 2
GSUBS = ((0, 104), (104, 96))
LANES = 16
CPR = EMBED_DIM // LANES
TRT = 1000
TRI = 512
BB = 64


def _sc_body(tab, idx1, idx2, pos, L2, pos_v,
             idxa0_v, idxb0_v, idxa1_v, idxb1_v,
             rowsa_v, rowsb_v, stg0_v, stg1_v,
             gsem, osem0, osem1, isem):
    wid = lax.axis_index("s") * NUM_CORES + lax.axis_index("c")
    wbase = wid * PPW
    idxa_vs = (idxa0_v, idxa1_v)
    idxb_vs = (idxb0_v, idxb1_v)
    stg_vs = (stg0_v, stg1_v)
    osems = (osem0, osem1)

    pltpu.sync_copy(pos, pos_v)

    def idx_copies(c, p):
        bb = wbase + c
        yield idx1.at[pl.ds(bb * 128, 128)], idxa_vs[p].at[pl.ds(0, 128)]
        yield idx2.at[pl.ds(bb * 128, 80)], idxa_vs[p].at[pl.ds(128, 80)]
        bb = BHALF + bb
        yield idx1.at[pl.ds(bb * 128, 128)], idxb_vs[p].at[pl.ds(0, 128)]
        yield idx2.at[pl.ds(bb * 128, 80)], idxb_vs[p].at[pl.ds(128, 80)]

    def start_idx(c, p):
        for src, dst in idx_copies(c, p):
            pltpu.async_copy(src, dst, isem)

    def wait_idx(c, p):
        for src, dst in idx_copies(c, p):
            pltpu.make_async_copy(src, dst, isem).wait()

    def start_gathers(p):
        for idx_v, rows_v in ((idxa_vs[p], rowsa_v), (idxb_vs[p], rowsb_v)):
            pltpu.async_copy(
                tab.at[idx_v.at[pl.ds(0, 128)]],
                rows_v.at[pl.ds(0, 128)], gsem)
            pltpu.async_copy(
                tab.at[idx_v.at[pl.ds(128, SEQ - 128)]],
                rows_v.at[pl.ds(128, SEQ - 128)], gsem)

    def wait_gathers():
        pltpu.make_async_copy(tab.at[pl.ds(0, SEQ)], rowsa_v, gsem).wait()
        pltpu.make_async_copy(tab.at[pl.ds(0, SEQ)], rowsb_v, gsem).wait()

    def start_out(c, b):
        pltpu.async_copy(
            stg_vs[b], L2.at[pl.ds((wbase + c) * SEQ, SEQ)], osems[b])

    def wait_out(c, b):
        pltpu.make_async_copy(
            stg_vs[b], L2.at[pl.ds((wbase + c) * SEQ, SEQ)],
            osems[b]).wait()

    def add_pos(b):
        stg_v = stg_vs[b]

        def row_body(r, _):
            for cc in range(CPR):
                sl = pl.ds(cc * LANES, LANES)
                p = pos_v[r, sl]
                stg_v[r, sl] = rowsa_v[r, sl] + p
                stg_v[r, pl.ds(EMBED_DIM + cc * LANES, LANES)] = (
                    rowsb_v[r, sl] + p)
            return 0

        lax.fori_loop(0, SEQ, row_body, 0)

    start_idx(0, 0)
    wait_idx(0, 0)
    start_gathers(0)

    def turn_body(k, _):
        for b in range(2):
            c = 2 * k + b
            p = b
            wait_gathers()

            @pl.when(c < PPW - 1)
            def _(c=c, p=p):
                start_idx(c + 1, 1 - p)

            @pl.when(c >= 2)
            def _(c=c, b=b):
                wait_out(c - 2, b)

            add_pos(b)
            start_out(c, b)

            @pl.when(c < PPW - 1)
            def _(c=c, p=p):
                wait_idx(c + 1, 1 - p)
                start_gathers(1 - p)

        return 0

    lax.fori_loop(0, NTURN, turn_body, 0)
    wait_out(PPW - 2, 0)
    wait_out(PPW - 1, 1)


def _tabpad_body(t_ref, o_ref):
    o_ref[:, :EMBED_DIM] = t_ref[...]


def _idxpad_body(i_ref, o1_ref, o2_ref):
    o1_ref[...] = i_ref[:, :128]
    o2_ref[:, :SEQ - 128] = i_ref[:, 128:]


def _depad_body(l_ref, o_ref):
    j = pl.program_id(1)

    @pl.when(j == 0)
    def _():
        o_ref[...] = l_ref[:, :EMBED_DIM].reshape(BB, SEQ, EMBED_DIM)

    @pl.when(j == 1)
    def _():
        o_ref[...] = l_ref[:, EMBED_DIM:].reshape(BB, SEQ, EMBED_DIM)


def _pad_tab(tab):
    return pl.pallas_call(
        _tabpad_body,
        grid=(VOCAB // TRT,),
        in_specs=[pl.BlockSpec((TRT, EMBED_DIM), lambda i: (i, 0))],
        out_specs=pl.BlockSpec((TRT, PAD_DIM), lambda i: (i, 0)),
        out_shape=jax.ShapeDtypeStruct((VOCAB, PAD_DIM), jnp.float32),
    )(tab)


def _depad(L2):
    return pl.pallas_call(
        _depad_body,
        grid=(BHALF // BB, 2),
        in_specs=[pl.BlockSpec((BB * SEQ, PAD_DIM), lambda i, j: (i, 0))],
        out_specs=pl.BlockSpec(
            (BB, SEQ, EMBED_DIM), lambda i, j: (j * (BHALF // BB) + i, 0, 0)),
        out_shape=jax.ShapeDtypeStruct((BATCH, SEQ, EMBED_DIM), jnp.float32),
    )(L2)


def _pad_idx(tokens):
    idx1, idx2 = pl.pallas_call(
        _idxpad_body,
        grid=(BATCH // TRI,),
        in_specs=[pl.BlockSpec((TRI, SEQ), lambda i: (i, 0))],
        out_specs=(pl.BlockSpec((TRI, 128), lambda i: (i, 0)),
                   pl.BlockSpec((TRI, 128), lambda i: (i, 0))),
        out_shape=(jax.ShapeDtypeStruct((BATCH, 128), jnp.int32),
                   jax.ShapeDtypeStruct((BATCH, 128), jnp.int32)),
    )(tokens.astype(jnp.int32))
    return idx1.reshape(BATCH * 128), idx2.reshape(BATCH * 128)


@jax.jit
def kernel(g_tok_table, e_tok_table, g_pos_table, e_pos_table,
           g_text_tokens, e_text_tokens):
    g_i1, g_i2 = _pad_idx(g_text_tokens)
    e_i1, e_i2 = _pad_idx(e_text_tokens)
    g_tab2 = _pad_tab(g_tok_table)
    e_tab2 = _pad_tab(e_tok_table)

    mesh = plsc.VectorSubcoreMesh(core_axis_name="c", subcore_axis_name="s")
    gather = functools.partial(
        pl.kernel,
        mesh=mesh,
        out_type=jax.ShapeDtypeStruct((HALF, PAD_DIM), jnp.float32),
        scratch_types=[
            pltpu.VMEM((SEQ, EMBED_DIM), jnp.float32),
        ] + [pltpu.VMEM((SEQ_PAD,), jnp.int32)] * 4
          + [pltpu.VMEM((SEQ, PAD_DIM), jnp.float32)] * 4
          + [pltpu.SemaphoreType.DMA] * 4,
    )(_sc_body)
    g_L2 = gather(g_tab2, g_i1, g_i2, g_pos_table)
    e_L2 = gather(e_tab2, e_i1, e_i2, e_pos_table)
    return (_depad(g_L2), _depad(e_L2))


# speedup vs baseline: 1.2034x; 1.2034x over previous
import functools

import jax
import jax.numpy as jnp
from jax import lax
from jax.experimental import pallas as pl
from jax.experimental.pallas import tpu as pltpu
from jax.experimental.pallas import tpu_sc as plsc

BATCH = 4096
SEQ = 200
EMBED_DIM = 64
ROWS = BATCH * SEQ
NUM_CORES = 2
NUM_SUBCORES = 16
NW = NUM_CORES * NUM_SUBCORES
RPW = ROWS // NW
SPC = 2
CHUNK = SPC * SEQ
NCHUNK = RPW // CHUNK
NBUF = 4
NITER = NCHUNK // NBUF
SUB = 80
NSUB = CHUNK // SUB
LANES = 16
CPR = EMBED_DIM // LANES


def _body(g_tab, e_tab, g_pos, e_pos, g_idx, e_idx, g_out, e_out,
          pos_v,
          idx0_v, idx1_v, idx2_v, idx3_v,
          rows0_v, rows1_v, rows2_v, rows3_v,
          gsem0, gsem1, gsem2, gsem3,
          osem0, osem1, osem2, osem3,
          isem0, isem1, isem2, isem3):
    wid = lax.axis_index("s") * NUM_CORES + lax.axis_index("c")
    base = wid * RPW
    idx_vs = (idx0_v, idx1_v, idx2_v, idx3_v)
    rows_vs = (rows0_v, rows1_v, rows2_v, rows3_v)
    gsems = (gsem0, gsem1, gsem2, gsem3)
    osems = (osem0, osem1, osem2, osem3)
    isems = (isem0, isem1, isem2, isem3)

    for tab, idx, out, pos in ((g_tab, g_idx, g_out, g_pos),
                               (e_tab, e_idx, e_out, e_pos)):
        pltpu.sync_copy(pos, pos_v)

        def start_idx(c, b, idx=idx):
            pltpu.async_copy(
                idx.at[pl.ds(base + c * CHUNK, CHUNK)], idx_vs[b], isems[b])

        def wait_idx(c, b, idx=idx):
            pltpu.make_async_copy(
                idx.at[pl.ds(base + c * CHUNK, CHUNK)], idx_vs[b],
                isems[b]).wait()

        def start_gathers(b, tab=tab):
            for j in range(NSUB):
                pltpu.async_copy(
                    tab.at[idx_vs[b].at[pl.ds(j * SUB, SUB)]],
                    rows_vs[b].at[pl.ds(j * SUB, SUB)],
                    gsems[b],
                )

        def wait_gathers(c, b, out=out):
            pltpu.make_async_copy(
                out.at[pl.ds(base + c * CHUNK, CHUNK)], rows_vs[b],
                gsems[b]).wait()

        def start_out(c, b, out=out):
            pltpu.async_copy(
                rows_vs[b], out.at[pl.ds(base + c * CHUNK, CHUNK)], osems[b])

        def wait_out(c, b, out=out):
            pltpu.make_async_copy(
                rows_vs[b], out.at[pl.ds(base + c * CHUNK, CHUNK)],
                osems[b]).wait()

        def add_pos(b):
            rows_v = rows_vs[b]

            def row_body(r, _):
                for cc in range(CPR):
                    sl = pl.ds(cc * LANES, LANES)
                    p = pos_v[r, sl]
                    for s in range(SPC):
                        row = s * SEQ + r
                        rows_v[row, sl] = rows_v[row, sl] + p
                return 0

            lax.fori_loop(0, SEQ, row_body, 0)

        for b in range(NBUF):
            start_idx(b, b)
        for b in range(NBUF):
            wait_idx(b, b)
            start_gathers(b)

        def turn_body(i, _):
            for b in range(NBUF):
                c = NBUF * i + b
                wait_gathers(c, b)

                @pl.when(i < NITER - 1)
                def _(c=c, b=b):
                    start_idx(c + NBUF, b)

                add_pos(b)
                start_out(c, b)

            @pl.when(i < NITER - 1)
            def _():
                for b in range(NBUF):
                    c = NBUF * i + b
                    wait_out(c, b)
                    wait_idx(c + NBUF, b)
                    start_gathers(b)

            return 0

        lax.fori_loop(0, NITER, turn_body, 0)
        for b in range(NBUF):
            wait_out(NCHUNK - NBUF + b, b)


@jax.jit
def kernel(g_tok_table, e_tok_table, g_pos_table, e_pos_table,
           g_text_tokens, e_text_tokens):
    g_idx = g_text_tokens.reshape(ROWS).astype(jnp.int32)
    e_idx = e_text_tokens.reshape(ROWS).astype(jnp.int32)

    mesh = plsc.VectorSubcoreMesh(core_axis_name="c", subcore_axis_name="s")
    run = functools.partial(
        pl.kernel,
        mesh=mesh,
        compiler_params=pltpu.CompilerParams(use_tc_tiling_on_sc=False),
        out_type=(
            jax.ShapeDtypeStruct((ROWS, EMBED_DIM), jnp.float32),
            jax.ShapeDtypeStruct((ROWS, EMBED_DIM), jnp.float32),
        ),
        scratch_types=[
            pltpu.VMEM((SEQ, EMBED_DIM), jnp.float32),
        ] + [pltpu.VMEM((CHUNK,), jnp.int32)] * NBUF
          + [pltpu.VMEM((CHUNK, EMBED_DIM), jnp.float32)] * NBUF
          + [pltpu.SemaphoreType.DMA] * (3 * NBUF),
    )(_body)
    g_out, e_out = run(g_tok_table, e_tok_table, g_pos_table, e_pos_table,
                       g_idx, e_idx)
    return (g_out.reshape(BATCH, SEQ, EMBED_DIM),
            e_out.reshape(BATCH, SEQ, EMBED_DIM))
